# Initial kernel scaffold; baseline (speedup 1.0000x reference)
import jax
import jax.numpy as jnp
from jax.experimental import pallas as pl


def kernel(x, edge_index, batch, W1, b1, W2, b2, Wmu, bmu, Wlv, blv):
    raise NotImplementedError("write your pallas kernel here")


# speedup vs baseline: 2.4195x; 2.4195x over previous
import functools

import jax
import jax.numpy as jnp
from jax.experimental import pallas as pl
from jax.experimental.pallas import tpu as pltpu


def _mm_kernel(x_ref, w_ref, o_ref):
    o_ref[...] = jnp.dot(x_ref[...], w_ref[...],
                         preferred_element_type=jnp.float32)


def _matmul(x, w, blk_m=400):
    m, k = x.shape
    k2, n = w.shape
    assert k == k2 and m % blk_m == 0
    grid = (m // blk_m,)
    return pl.pallas_call(
        _mm_kernel,
        grid=grid,
        in_specs=[
            pl.BlockSpec((blk_m, k), lambda i: (i, 0)),
            pl.BlockSpec((k, n), lambda i: (0, 0)),
        ],
        out_specs=pl.BlockSpec((blk_m, n), lambda i: (i, 0)),
        out_shape=jax.ShapeDtypeStruct((m, n), jnp.float32),
    )(x, w)


def kernel(x, edge_index, batch, W1, b1, W2, b2, Wmu, bmu, Wlv, blv):
    N = x.shape[0]
    src = edge_index[0]
    dst = edge_index[1]
    deg = jnp.ones((N,), jnp.float32).at[dst].add(1.0)
    dis = jax.lax.rsqrt(deg)

    def conv(h, W, b):
        y = _matmul(h, W) * dis[:, None]
        acc = y.at[dst].add(y[src])
        return jax.nn.relu(acc * dis[:, None] + b)

    h = conv(x, W1, b1)
    h = conv(h, W2, b2)

    G = 64
    sums = jax.ops.segment_sum(h, batch, num_segments=G)
    counts = jax.ops.segment_sum(jnp.ones((N,), jnp.float32), batch,
                                 num_segments=G)
    hg = sums / jnp.maximum(counts, 1.0)[:, None]
    mu = _matmul(hg, Wmu, blk_m=64) + bmu
    logvar = _matmul(hg, Wlv, blk_m=64) + blv
    return (mu, logvar)


# speedup vs baseline: 6.1786x; 2.5537x over previous
import functools

import jax
import jax.numpy as jnp
from jax import lax
from jax.experimental import pallas as pl
from jax.experimental.pallas import tpu as pltpu
from jax.experimental.pallas import tpu_sc as plsc

NC = 2
NS = 16
LN = 16

N = 10000
E = 160000
NP = 10240
EP = 163840
C = 128
D = 256
HD = 256
HH = 128
G = 64
L = 64

_MESH = plsc.VectorSubcoreMesh(core_axis_name="c", subcore_axis_name="s")


@functools.partial(
    pl.kernel,
    mesh=_MESH,
    out_type=jax.ShapeDtypeStruct((NP, 16), jnp.float32),
    scratch_types=[
        pltpu.VMEM_SHARED((NP, 16), jnp.float32),
        pltpu.VMEM((C,), jnp.int32),
        pltpu.VMEM((C, 16), jnp.float32),
        pltpu.VMEM((C, 16), jnp.float32),
    ],
)
def _deg_sc(dst_hbm, ones_hbm, deg_hbm, acc_sh, dst_v, ones_v, buf_v):
    cid = lax.axis_index("c")
    sid = lax.axis_index("s")

    @pl.when(cid == 0)
    def _():
        pltpu.sync_copy(ones_hbm, ones_v)
        for k in range(NP // NS // C):
            pltpu.sync_copy(ones_v, acc_sh.at[pl.ds(sid * (NP // NS) + k * C, C)])
        plsc.subcore_barrier()
        base = sid * (EP // NS)

        def body(i, _):
            pltpu.sync_copy(dst_hbm.at[pl.ds(base + i * C, C)], dst_v)
            pltpu.sync_copy(ones_v, acc_sh.at[dst_v], add=True)
            return 0

        lax.fori_loop(0, EP // NS // C, body, 0)
        plsc.subcore_barrier()
        for k in range(NP // NS // C):
            off = sid * (NP // NS) + k * C
            pltpu.sync_copy(acc_sh.at[pl.ds(off, C)], buf_v)
            pltpu.sync_copy(buf_v, deg_hbm.at[pl.ds(off, C)])


@functools.partial(
    pl.kernel,
    mesh=_MESH,
    out_type=jax.ShapeDtypeStruct((2 * NP, HH), jnp.float32),
    scratch_types=[
        pltpu.VMEM_SHARED((NP, HH), jnp.float32),
        pltpu.VMEM((C,), jnp.int32),
        pltpu.VMEM((C,), jnp.int32),
        pltpu.VMEM((C, HH), jnp.float32),
        pltpu.SemaphoreType.DMA,
    ],
)
def _spmm_sc(y_hbm, src_hbm, dst_hbm, out_hbm, acc_sh, src_v, dst_v, rows_v, sem):
    cid = lax.axis_index("c")
    sid = lax.axis_index("s")
    row0 = cid * NP

    for k in range(NP // NS // C):
        off = sid * (NP // NS) + k * C
        pltpu.sync_copy(y_hbm.at[pl.ds(row0 + off, C)], rows_v)
        pltpu.sync_copy(rows_v, acc_sh.at[pl.ds(off, C)])
    plsc.subcore_barrier()

    base = sid * (EP // NS)

    def body(i, _):
        pltpu.sync_copy(src_hbm.at[pl.ds(base + i * C, C)], src_v)
        pltpu.sync_copy(dst_hbm.at[pl.ds(base + i * C, C)], dst_v)
        for j in range(C // LN):
            sl = pl.ds(j * LN, LN)
            src_v[sl] = src_v[sl] + row0
        pltpu.async_copy(y_hbm.at[src_v], rows_v, sem).wait()
        pltpu.sync_copy(rows_v, acc_sh.at[dst_v], add=True)
        return 0

    lax.fori_loop(0, EP // NS // C, body, 0)
    plsc.subcore_barrier()

    for k in range(NP // NS // C):
        off = sid * (NP // NS) + k * C
        pltpu.sync_copy(acc_sh.at[pl.ds(off, C)], rows_v)
        pltpu.sync_copy(rows_v, out_hbm.at[pl.ds(row0 + off, C)])


BM = 512


def _mm1_body(x_ref, w_ref, deg_ref, o_ref):
    dis = lax.rsqrt(deg_ref[:, :1])
    o_ref[...] = jnp.dot(x_ref[...], w_ref[...],
                         preferred_element_type=jnp.float32) * dis


def _mm1(x_pad, W1, deg):
    return pl.pallas_call(
        _mm1_body,
        grid=(2, NP // BM),
        in_specs=[
            pl.BlockSpec((BM, D), lambda c, i: (i, 0)),
            pl.BlockSpec((D, HH), lambda c, i: (0, c)),
            pl.BlockSpec((BM, 16), lambda c, i: (i, 0)),
        ],
        out_specs=pl.BlockSpec((BM, HH), lambda c, i: (c * (NP // BM) + i, 0)),
        out_shape=jax.ShapeDtypeStruct((2 * NP, HH), jnp.float32),
    )(x_pad, W1, deg)


def _mm2_body(a_ref, w_ref, deg_ref, b_ref, o_ref):
    dis = lax.rsqrt(deg_ref[:, :1])
    h = jnp.concatenate([a_ref[0], a_ref[1]], axis=1)
    h = jax.nn.relu(h * dis + b_ref[...])
    y = jnp.dot(h, w_ref[...], preferred_element_type=jnp.float32) * dis
    o_ref[0] = y[:, :HH]
    o_ref[1] = y[:, HH:]


def _mm2(acc3, W2, deg, b1):
    return pl.pallas_call(
        _mm2_body,
        grid=(NP // BM,),
        in_specs=[
            pl.BlockSpec((2, BM, HH), lambda i: (0, i, 0)),
            pl.BlockSpec((HD, HD), lambda i: (0, 0)),
            pl.BlockSpec((BM, 16), lambda i: (i, 0)),
            pl.BlockSpec((1, HD), lambda i: (0, 0)),
        ],
        out_specs=pl.BlockSpec((2, BM, HH), lambda i: (0, i, 0)),
        out_shape=jax.ShapeDtypeStruct((2, NP, HH), jnp.float32),
    )(acc3, W2, deg, b1)


def _final_body(a_ref, deg_ref, b_ref, bt_ref, wmu_ref, bmu_ref, wlv_ref,
                blv_ref, mu_ref, lv_ref, sums, counts):
    i = pl.program_id(0)

    @pl.when(i == 0)
    def _():
        sums[...] = jnp.zeros_like(sums)
        counts[...] = jnp.zeros_like(counts)

    dis = lax.rsqrt(deg_ref[:, :1])
    h = jnp.concatenate([a_ref[0], a_ref[1]], axis=1)
    h = jax.nn.relu(h * dis + b_ref[...])
    ids = lax.broadcasted_iota(jnp.int32, (G, BM), 0)
    oh = (ids == jnp.reshape(bt_ref[...], (1, BM))).astype(jnp.float32)
    sums[...] += jnp.dot(oh, h, preferred_element_type=jnp.float32)
    counts[...] += jnp.broadcast_to(
        jnp.sum(oh, axis=1, keepdims=True), counts.shape)

    @pl.when(i == pl.num_programs(0) - 1)
    def _():
        hg = sums[...] / jnp.maximum(counts[:, :1], 1.0)
        mu_ref[...] = jnp.dot(hg, wmu_ref[...],
                              preferred_element_type=jnp.float32) + bmu_ref[...]
        lv_ref[...] = jnp.dot(hg, wlv_ref[...],
                              preferred_element_type=jnp.float32) + blv_ref[...]


def _final(acc3, deg, b2, batch2d, Wmu, bmu, Wlv, blv):
    return pl.pallas_call(
        _final_body,
        grid=(NP // BM,),
        in_specs=[
            pl.BlockSpec((2, BM, HH), lambda i: (0, i, 0)),
            pl.BlockSpec((BM, 16), lambda i: (i, 0)),
            pl.BlockSpec((1, HD), lambda i: (0, 0)),
            pl.BlockSpec((BM, 1), lambda i: (i, 0)),
            pl.BlockSpec((HD, L), lambda i: (0, 0)),
            pl.BlockSpec((1, L), lambda i: (0, 0)),
            pl.BlockSpec((HD, L), lambda i: (0, 0)),
            pl.BlockSpec((1, L), lambda i: (0, 0)),
        ],
        out_specs=[
            pl.BlockSpec((G, L), lambda i: (0, 0)),
            pl.BlockSpec((G, L), lambda i: (0, 0)),
        ],
        out_shape=[
            jax.ShapeDtypeStruct((G, L), jnp.float32),
            jax.ShapeDtypeStruct((G, L), jnp.float32),
        ],
        scratch_shapes=[
            pltpu.VMEM((G, HD), jnp.float32),
            pltpu.VMEM((G, 128), jnp.float32),
        ],
    )(acc3, deg, b2, batch2d, Wmu, bmu, Wlv, blv)


def kernel(x, edge_index, batch, W1, b1, W2, b2, Wmu, bmu, Wlv, blv):
    x_pad = jnp.zeros((NP, D), jnp.float32).at[:N].set(x)
    padi = jnp.full((EP - E,), N, jnp.int32)
    srcp = jnp.concatenate([edge_index[0], padi])
    dstp = jnp.concatenate([edge_index[1], padi])
    batch2d = jnp.concatenate(
        [batch, jnp.full((NP - N,), G, jnp.int32)])[:, None]
    ones = jnp.ones((C, 16), jnp.float32)
    b1r = b1[None, :]
    b2r = b2[None, :]
    bmur = bmu[None, :]
    blvr = blv[None, :]

    deg = _deg_sc(dstp, ones)
    y1 = _mm1(x_pad, W1, deg)
    acc1 = _spmm_sc(y1, srcp, dstp)
    y2 = _mm2(acc1.reshape(2, NP, HH), W2, deg, b1r)
    acc2 = _spmm_sc(y2.reshape(2 * NP, HH), srcp, dstp)
    mu, lv = _final(acc2.reshape(2, NP, HH), deg, b2r, batch2d,
                    Wmu, bmur, Wlv, blvr)
    return (mu, lv)
